# baseline (speedup 1.0000x reference)
import jax
import jax.numpy as jnp
from jax import lax
from jax.experimental import pallas as pl
from jax.experimental.pallas import tpu as pltpu
from jax.experimental.pallas import tpu_sc as plsc

_M = 4096
_A = 50
_AP = 56
_D = 128

_NC = 2
_NS = 16
_NW = _NC * _NS

_MPW = _M // _NW
_NBUF = 8
_NGROUP = _MPW // _NBUF


def _make_sc_gather():
    mesh = plsc.VectorSubcoreMesh(core_axis_name="c", subcore_axis_name="s")

    def body(tab_hbm, x_hbm, out_hbm, idx_v, stages, gsems, osems):
        wid = lax.axis_index("s") * _NC + lax.axis_index("c")
        wmol = wid * _MPW

        pltpu.sync_copy(x_hbm.at[pl.ds(wmol, _MPW)], idx_v)

        def issue(b, i):
            return pltpu.async_copy(
                tab_hbm.at[idx_v.at[i]], stages[b], gsems[b]
            )

        def write_out(b, i, g):
            g.wait()
            return pltpu.async_copy(
                stages[b], out_hbm.at[wmol + i], osems[b]
            )

        def wait_out(b):
            pltpu.make_async_copy(
                stages[b], out_hbm.at[wmol], osems[b]
            ).wait()

        descs = [issue(b, b) for b in range(_NBUF)]
        for b in range(_NBUF):
            write_out(b, b, descs[b])

        def grp(g, carry):
            ds_ = []
            for b in range(_NBUF):
                wait_out(b)
                ds_.append(issue(b, g * _NBUF + b))
            for b in range(_NBUF):
                write_out(b, g * _NBUF + b, ds_[b])
            return carry

        lax.fori_loop(1, _NGROUP, grp, 0)

        for b in range(_NBUF):
            wait_out(b)

    return pl.kernel(
        body,
        out_type=jax.ShapeDtypeStruct((_M, _AP, _D), jnp.float32),
        mesh=mesh,
        scratch_types=[
            pltpu.VMEM((_MPW, _AP), jnp.int32),
            [pltpu.VMEM((_AP, _D), jnp.float32) for _ in range(_NBUF)],
            [pltpu.SemaphoreType.DMA for _ in range(_NBUF)],
            [pltpu.SemaphoreType.DMA for _ in range(_NBUF)],
        ],
    )


_sc_gather = _make_sc_gather()


@jax.jit
def kernel(x, positions, token_emb):
    xp = jnp.pad(x.astype(jnp.int32), ((0, 0), (0, _AP - _A)))
    emb = _sc_gather(token_emb, xp)
    return jnp.concatenate([emb[:, : _A, :], positions], axis=-1)


# speedup vs baseline: 3.5505x; 3.5505x over previous
import jax
import jax.numpy as jnp
from jax import lax
from jax.experimental import pallas as pl
from jax.experimental.pallas import tpu as pltpu
from jax.experimental.pallas import tpu_sc as plsc

_M = 4096
_A = 50
_D = 128
_DP = 131
_B = _M * _A

_NC = 2
_NS = 16
_NW = _NC * _NS

_RPW = _B // _NW
_CH = 200
_NBUF = 4
_NCHUNK = _RPW // _CH
_NGROUP = _NCHUNK // _NBUF

_BM = 4


def _make_sc_gather():
    mesh = plsc.VectorSubcoreMesh(core_axis_name="c", subcore_axis_name="s")

    def body(tab_hbm, idx_hbm, out_hbm, idx_v, stages, gsems, osems):
        wid = lax.axis_index("s") * _NC + lax.axis_index("c")
        wbase = wid * _RPW

        pltpu.sync_copy(idx_hbm.at[pl.ds(wbase, _RPW)], idx_v)

        def issue(b, i):
            return pltpu.async_copy(
                tab_hbm.at[idx_v.at[pl.ds(i * _CH, _CH)]],
                stages[b],
                gsems[b],
            )

        def write_out(b, i, g):
            g.wait()
            return pltpu.async_copy(
                stages[b], out_hbm.at[pl.ds(wbase + i * _CH, _CH)], osems[b]
            )

        def wait_out(b):
            pltpu.make_async_copy(
                stages[b], out_hbm.at[pl.ds(wbase, _CH)], osems[b]
            ).wait()

        descs = [issue(b, b) for b in range(_NBUF)]
        for b in range(_NBUF):
            write_out(b, b, descs[b])

        def grp(g, carry):
            ds_ = []
            for b in range(_NBUF):
                wait_out(b)
                ds_.append(issue(b, g * _NBUF + b))
            for b in range(_NBUF):
                write_out(b, g * _NBUF + b, ds_[b])
            return carry

        lax.fori_loop(1, _NGROUP, grp, 0)

        for b in range(_NBUF):
            wait_out(b)

    return pl.kernel(
        body,
        out_type=jax.ShapeDtypeStruct((_B, _D), jnp.float32),
        mesh=mesh,
        scratch_types=[
            pltpu.VMEM((_RPW,), jnp.int32),
            [pltpu.VMEM((_CH, _D), jnp.float32) for _ in range(_NBUF)],
            [pltpu.SemaphoreType.DMA for _ in range(_NBUF)],
            [pltpu.SemaphoreType.DMA for _ in range(_NBUF)],
        ],
    )


_sc_gather = _make_sc_gather()


@jax.jit
def kernel(x, positions, token_emb):
    idx = x.astype(jnp.int32).reshape(_B)
    pos_flat = positions.reshape(_B, 3)
    emb = _sc_gather(token_emb, idx)
    return jnp.concatenate(
        [emb.reshape(_M, _A, _D), pos_flat.reshape(_M, _A, 3)], axis=-1
    )
